# Initial kernel scaffold; baseline (speedup 1.0000x reference)
import jax
import jax.numpy as jnp
from jax.experimental import pallas as pl


def kernel(vertices, faces):
    raise NotImplementedError("write your pallas kernel here")


# speedup vs baseline: 3.0652x; 3.0652x over previous
import jax
import jax.numpy as jnp
from jax import lax
from jax.experimental import pallas as pl
from jax.experimental.pallas import tpu as pltpu
from jax.experimental.pallas import tpu_sc as plsc

NV = 100000
NF = 200000
L = 16
NC, NS = 2, 16
NW = NC * NS

CH = 128
FW = 6272
FW3 = 3 * FW
NCH = FW3 // CH
NSTEP = FW // L
NSTEP3 = FW3 // L
VB = 3136
LAG = 8

_EDGE = ((0, 1), (1, 2), (2, 0))


def _body(verts_hbm, faces_hbm, overts_hbm, ofaces_hbm,
          e0, e1, e2, g0, g1, g2, sem):
    wid = lax.axis_index("s") * NC + lax.axis_index("c")
    fbase = jnp.minimum(wid * FW, NF - FW)
    vbase = jnp.minimum(wid * VB, NV - VB)

    iota = lax.iota(jnp.int32, L)

    pltpu.sync_copy(verts_hbm.at[pl.ds(3 * vbase, 3 * VB)],
                    g0.at[pl.ds(0, 3 * VB)])
    pltpu.sync_copy(g0.at[pl.ds(0, 3 * VB)],
                    overts_hbm.at[pl.ds(3 * vbase, 3 * VB)])

    pltpu.sync_copy(faces_hbm.at[pl.ds(3 * fbase, FW3)], e2)

    def _build(c, dst):
        def step(i, _):
            kv = iota + i * L
            jv = kv // 3
            dv = kv - jv * 3
            f = plsc.load_gather(e2, [kv - dv + c])
            dst[pl.ds(i * L, L)] = f * 3 + dv
            return 0
        lax.fori_loop(0, NSTEP3, step, 0)

    _build(0, e0)
    _build(1, e1)
    _build(2, e2)

    def _fire(m, eb, gb):
        off = m * CH
        return pltpu.async_copy(
            verts_hbm.at[eb.at[pl.ds(off, CH)]], gb.at[pl.ds(off, CH)], sem)

    def _gather(m, _):
        @pl.when(m < NCH)
        def _():
            for eb, gb in ((e0, g0), (e1, g1), (e2, g2)):
                _fire(m, eb, gb)

        @pl.when(m >= LAG)
        def _():
            for eb, gb in ((e0, g0), (e1, g1), (e2, g2)):
                pltpu.make_async_copy(
                    verts_hbm.at[eb.at[pl.ds(0, CH)]],
                    gb.at[pl.ds(0, CH)], sem).wait()
        return 0

    lax.fori_loop(0, NCH + LAG, _gather, 0)

    def _mean(i, _):
        s = pl.ds(i * L, L)
        g0[s] = (g0[s] + g1[s] + g2[s]) / 3.0
        return 0

    lax.fori_loop(0, NSTEP3, _mean, 0)
    pltpu.sync_copy(g0, overts_hbm.at[pl.ds(3 * (NV + fbase), FW3)])

    pltpu.sync_copy(faces_hbm.at[pl.ds(3 * fbase, FW3)], e0)
    for b, (p0, p1) in enumerate(_EDGE):
        def _faces(i, _, p0=p0, p1=p1):
            j = iota + i * L
            j3 = j * 3
            v0 = plsc.load_gather(e0, [j3 + p0])
            v1 = plsc.load_gather(e0, [j3 + p1])
            plsc.store_scatter(e1, [j3], v0)
            plsc.store_scatter(e1, [j3 + 1], v1)
            plsc.store_scatter(e1, [j3 + 2], NV + fbase + j)
            return 0

        lax.fori_loop(0, NSTEP, _faces, 0)
        pltpu.sync_copy(e1, ofaces_hbm.at[pl.ds(3 * (b * NF + fbase), FW3)])


@jax.jit
def _upsample(verts_flat, faces_flat):
    mesh = plsc.VectorSubcoreMesh(
        core_axis_name="c", subcore_axis_name="s",
        num_cores=NC, num_subcores=NS)
    fn = pl.kernel(
        _body,
        out_type=(
            jax.ShapeDtypeStruct((3 * (NV + NF),), jnp.float32),
            jax.ShapeDtypeStruct((3 * NF * 3,), jnp.int32),
        ),
        mesh=mesh,
        compiler_params=pltpu.CompilerParams(
            needs_layout_passes=False, use_tc_tiling_on_sc=False),
        scratch_types=[
            pltpu.VMEM((FW3,), jnp.int32),
            pltpu.VMEM((FW3,), jnp.int32),
            pltpu.VMEM((FW3,), jnp.int32),
            pltpu.VMEM((FW3,), jnp.float32),
            pltpu.VMEM((FW3,), jnp.float32),
            pltpu.VMEM((FW3,), jnp.float32),
            pltpu.SemaphoreType.DMA,
        ],
    )
    overts_flat, ofaces_flat = fn(verts_flat, faces_flat)
    return overts_flat.reshape(NV + NF, 3), ofaces_flat.reshape(3 * NF, 3)


def kernel(vertices, faces):
    return _upsample(vertices.reshape(-1),
                     faces.astype(jnp.int32).reshape(-1))


# speedup vs baseline: 3.8782x; 1.2652x over previous
import jax
import jax.numpy as jnp
from jax import lax
from jax.experimental import pallas as pl
from jax.experimental.pallas import tpu as pltpu
from jax.experimental.pallas import tpu_sc as plsc

NV = 100000
NF = 200000
L = 16
NC, NS = 2, 16
NW = NC * NS

CH = 128
FW = 6272
FW3 = 3 * FW
NCH = FW3 // CH
NSTEP = FW // L
NSTEP3 = FW3 // L
VB = 3136
LAG = 8

_EDGE = ((0, 1), (1, 2), (2, 0))

_MESH = plsc.VectorSubcoreMesh(
    core_axis_name="c", subcore_axis_name="s",
    num_cores=NC, num_subcores=NS)
_PARAMS = pltpu.CompilerParams(
    needs_layout_passes=False, use_tc_tiling_on_sc=False)


def _wid():
    return lax.axis_index("s") * NC + lax.axis_index("c")


def _faces_body(faces_hbm, ofaces_hbm, fbuf, obuf, sem):
    fbase = jnp.minimum(_wid() * FW, NF - FW)
    iota = lax.iota(jnp.int32, L)

    pltpu.sync_copy(faces_hbm.at[pl.ds(3 * fbase, FW3)], fbuf)
    for b, (p0, p1) in enumerate(_EDGE):
        def _faces(i, _, p0=p0, p1=p1):
            j = iota + i * L
            j3 = j * 3
            v0 = plsc.load_gather(fbuf, [j3 + p0])
            v1 = plsc.load_gather(fbuf, [j3 + p1])
            plsc.store_scatter(obuf, [j3], v0)
            plsc.store_scatter(obuf, [j3 + 1], v1)
            plsc.store_scatter(obuf, [j3 + 2], NV + fbase + j)
            return 0

        lax.fori_loop(0, NSTEP, _faces, 0)
        pltpu.sync_copy(obuf, ofaces_hbm.at[pl.ds(3 * (b * NF + fbase), FW3)])


def _centroid_body(verts_hbm, faces_hbm, overts_hbm, e0, e1, e2, g0, g1, g2,
                   sem):
    fbase = jnp.minimum(_wid() * FW, NF - FW)
    vbase = jnp.minimum(_wid() * VB, NV - VB)
    iota = lax.iota(jnp.int32, L)

    pltpu.sync_copy(verts_hbm.at[pl.ds(3 * vbase, 3 * VB)],
                    g0.at[pl.ds(0, 3 * VB)])
    pltpu.sync_copy(g0.at[pl.ds(0, 3 * VB)],
                    overts_hbm.at[pl.ds(3 * vbase, 3 * VB)])

    pltpu.sync_copy(faces_hbm.at[pl.ds(3 * fbase, FW3)], e2)

    def _build(c, dst):
        def step(i, _):
            kv = iota + i * L
            jv = kv // 3
            dv = kv - jv * 3
            f = plsc.load_gather(e2, [kv - dv + c])
            dst[pl.ds(i * L, L)] = f * 3 + dv
            return 0
        lax.fori_loop(0, NSTEP3, step, 0)

    _build(0, e0)
    _build(1, e1)
    _build(2, e2)

    def _gather(m, _):
        @pl.when(m < NCH)
        def _():
            for eb, gb in ((e0, g0), (e1, g1), (e2, g2)):
                off = m * CH
                pltpu.async_copy(
                    verts_hbm.at[eb.at[pl.ds(off, CH)]],
                    gb.at[pl.ds(off, CH)], sem)

        @pl.when(m >= LAG)
        def _():
            for eb, gb in ((e0, g0), (e1, g1), (e2, g2)):
                pltpu.make_async_copy(
                    verts_hbm.at[eb.at[pl.ds(0, CH)]],
                    gb.at[pl.ds(0, CH)], sem).wait()
        return 0

    lax.fori_loop(0, NCH + LAG, _gather, 0)

    def _mean(i, _):
        s = pl.ds(i * L, L)
        g0[s] = (g0[s] + g1[s] + g2[s]) / 3.0
        return 0

    lax.fori_loop(0, NSTEP3, _mean, 0)
    pltpu.sync_copy(g0, overts_hbm.at[pl.ds(3 * (NV + fbase), FW3)])


@jax.jit
def _upsample(verts_flat, faces_flat):
    faces_fn = pl.kernel(
        _faces_body,
        out_type=jax.ShapeDtypeStruct((3 * NF * 3,), jnp.int32),
        mesh=_MESH,
        compiler_params=_PARAMS,
        scratch_types=[
            pltpu.VMEM((FW3,), jnp.int32),
            pltpu.VMEM((FW3,), jnp.int32),
            pltpu.SemaphoreType.DMA,
        ],
    )
    cen_fn = pl.kernel(
        _centroid_body,
        out_type=jax.ShapeDtypeStruct((3 * (NV + NF),), jnp.float32),
        mesh=_MESH,
        compiler_params=_PARAMS,
        scratch_types=[
            pltpu.VMEM((FW3,), jnp.int32),
            pltpu.VMEM((FW3,), jnp.int32),
            pltpu.VMEM((FW3,), jnp.int32),
            pltpu.VMEM((FW3,), jnp.float32),
            pltpu.VMEM((FW3,), jnp.float32),
            pltpu.VMEM((FW3,), jnp.float32),
            pltpu.SemaphoreType.DMA,
        ],
    )
    ofaces_flat = faces_fn(faces_flat)
    overts_flat = cen_fn(verts_flat, faces_flat)
    return overts_flat.reshape(NV + NF, 3), ofaces_flat.reshape(3 * NF, 3)


def kernel(vertices, faces):
    return _upsample(vertices.reshape(-1),
                     faces.astype(jnp.int32).reshape(-1))
